# baseline (speedup 1.0000x reference)
import functools

import jax
import jax.numpy as jnp
from jax import lax
from jax.experimental import pallas as pl
from jax.experimental.pallas import tpu as pltpu
from jax.experimental.pallas import tpu_sc as plsc

N = 10000
E = 320000
D = 128
L = 3
G = 512
T = 128

NC = 2
NS = 16
NW = NC * NS

NP = 10240
EP = 327680
GP = 640
NWROWS = NP // NW
EW = EP // NW
CN = 80
CE = 80
GROWS = GP // NS
NROWS_T = NP // NS

BR = 256
NBLK = NP // BR

_mesh = plsc.VectorSubcoreMesh(core_axis_name="c", subcore_axis_name="s")


def _zero_rows(buf, nrows):
    def z(r, _):
        for j in range(8):
            buf[r, pl.ds(j * 16, 16)] = jnp.zeros((16,), jnp.float32)
        return 0
    lax.fori_loop(0, nrows, z, 0)


def _make_node_kernel(affine, relu, vne_gather, vne_bcast, pool, write_hl,
                      counts):
    out_type = []
    if write_hl:
        out_type.append(jax.ShapeDtypeStruct((NP, D), jnp.float32))
    if pool:
        out_type.append(jax.ShapeDtypeStruct((NC, GP, D), jnp.float32))
    if counts:
        out_type.append(jax.ShapeDtypeStruct((NC, GP, D), jnp.float32))

    scratch = [pltpu.VMEM((CN, D), jnp.float32),
               pltpu.VMEM((CN, D), jnp.float32),
               pltpu.VMEM((CN,), jnp.int32),
               pltpu.VMEM((CN,), jnp.int32),
               pltpu.VMEM((2, D), jnp.float32),
               pltpu.VMEM((1, D), jnp.float32),
               pltpu.VMEM((CN, D), jnp.float32),
               pltpu.SemaphoreType.DMA,
               pltpu.SemaphoreType.DMA]
    if pool:
        scratch.append(pltpu.VMEM_SHARED((GP, D), jnp.float32))
    if counts:
        scratch.append(pltpu.VMEM_SHARED((GP, D), jnp.float32))

    def body(*refs):
        it = iter(refs)
        t_hbm = next(it)
        aff_hbm = next(it) if affine else None
        vne_hbm = next(it) if (vne_gather or vne_bcast) else None
        bg_hbm = next(it) if vne_gather else None
        bp_hbm = next(it) if (pool or counts) else None
        hl_hbm = next(it) if write_hl else None
        pooled_hbm = next(it) if pool else None
        counts_hbm = next(it) if counts else None
        buf = next(it); buf2 = next(it); bgv = next(it); bpv = next(it)
        affv = next(it); vnev = next(it); onesv = next(it)
        sem = next(it); sem2 = next(it)
        acc = next(it) if pool else None
        acc2 = next(it) if counts else None

        cid = lax.axis_index("c")
        sid = lax.axis_index("s")
        wid = sid * NC + cid

        if pool or counts:
            _zero_rows(buf, GROWS)
            if pool:
                pltpu.sync_copy(buf.at[pl.ds(0, GROWS)],
                                acc.at[pl.ds(sid * GROWS, GROWS)])
            if counts:
                pltpu.sync_copy(buf.at[pl.ds(0, GROWS)],
                                acc2.at[pl.ds(sid * GROWS, GROWS)])

                def o(r, _):
                    for j in range(8):
                        onesv[r, pl.ds(j * 16, 16)] = jnp.ones((16,),
                                                               jnp.float32)
                    return 0
                lax.fori_loop(0, CN, o, 0)
            plsc.subcore_barrier()

        if affine:
            pltpu.sync_copy(aff_hbm, affv)
        if vne_bcast:
            pltpu.sync_copy(vne_hbm, vnev)

        scale_v = ([affv[0, pl.ds(j * 16, 16)] for j in range(8)]
                   if affine else None)
        shift_v = ([affv[1, pl.ds(j * 16, 16)] for j in range(8)]
                   if affine else None)
        vrow_v = ([vnev[0, pl.ds(j * 16, 16)] for j in range(8)]
                  if vne_bcast else None)

        def chunk(k, _):
            base = wid * NWROWS + k * CN
            pltpu.sync_copy(t_hbm.at[pl.ds(base, CN)], buf)
            if vne_gather:
                pltpu.sync_copy(bg_hbm.at[pl.ds(base, CN)], bgv)
            if pool or counts:
                pltpu.sync_copy(bp_hbm.at[pl.ds(base, CN)], bpv)
            if vne_gather:
                pltpu.async_copy(vne_hbm.at[bgv], buf2, sem).wait()

            def row(r, _):
                for j in range(8):
                    sl = pl.ds(j * 16, 16)
                    v = buf[r, sl]
                    if affine:
                        v = v * scale_v[j] + shift_v[j]
                    if relu:
                        v = jnp.maximum(v, 0.0)
                    if vne_bcast:
                        v = v + vrow_v[j]
                    if vne_gather:
                        v = v + buf2[r, sl]
                    buf[r, sl] = v
                return 0
            lax.fori_loop(0, CN, row, 0)

            if write_hl:
                pltpu.sync_copy(buf, hl_hbm.at[pl.ds(base, CN)])
            if pool:
                pltpu.sync_copy(buf, acc.at[bpv], add=True)
            if counts:
                pltpu.sync_copy(onesv, acc2.at[bpv], add=True)
            return 0
        lax.fori_loop(0, NWROWS // CN, chunk, 0)

        if pool or counts:
            plsc.subcore_barrier()
        if pool:
            pltpu.sync_copy(acc.at[pl.ds(sid * GROWS, GROWS)],
                            pooled_hbm.at[cid, pl.ds(sid * GROWS, GROWS)])
        if counts:
            pltpu.sync_copy(acc2.at[pl.ds(sid * GROWS, GROWS)],
                            counts_hbm.at[cid, pl.ds(sid * GROWS, GROWS)])

    return pl.kernel(body, out_type=out_type, mesh=_mesh,
                     scratch_types=scratch)


NCH = EW // CE
NZB = 128


def _edge_body(hl_hbm, ftab_hbm, src_hbm, dst_hbm, eix_hbm, aggr_hbm,
               bh0, bh1, bf0, bf1,
               sv0, sv1, sv2, sv3, dv0, dv1, dv2, dv3, ev0, ev1, ev2, ev3,
               g0, g1, s0, s1, i0, i1, i2, i3, acc):
    cid = lax.axis_index("c")
    sid = lax.axis_index("s")
    wid = sid * NC + cid

    bufh = (bh0, bh1)
    buff = (bf0, bf1)
    svs = (sv0, sv1, sv2, sv3)
    dvs = (dv0, dv1, dv2, dv3)
    evs = (ev0, ev1, ev2, ev3)
    gsem = (g0, g1)
    ssem = (s0, s1)
    isem = (i0, i1, i2, i3)

    _zero_rows(bh0, CE)
    for i in range(NROWS_T // CE):
        pltpu.sync_copy(bh0, acc.at[pl.ds(sid * NROWS_T + i * CE, CE)])
    plsc.subcore_barrier()

    def iissue(r, k):
        pltpu.async_copy(src_hbm.at[wid, k], svs[r], isem[r])
        pltpu.async_copy(dst_hbm.at[wid, k], dvs[r], isem[r])
        pltpu.async_copy(eix_hbm.at[wid, k], evs[r], isem[r])

    def iwait(r):
        for ref in (svs[r], dvs[r], evs[r]):
            pltpu.make_async_copy(src_hbm.at[wid, 0], ref, isem[r]).wait()

    def gissue(p, r):
        pltpu.async_copy(hl_hbm.at[svs[r]], bufh[p], gsem[p])
        pltpu.async_copy(ftab_hbm.at[evs[r]], buff[p], gsem[p])

    def gwait(p):
        pltpu.make_async_copy(hl_hbm.at[svs[0]], bufh[p], gsem[p]).wait()
        pltpu.make_async_copy(hl_hbm.at[svs[0]], buff[p], gsem[p]).wait()

    def sissue(p, r):
        pltpu.async_copy(bufh[p], acc.at[dvs[r]], ssem[p], add=True)

    def swait(p):
        pltpu.make_async_copy(bufh[p], acc.at[dvs[0]], ssem[p]).wait()

    def compute(p):
        bh, bf = bufh[p], buff[p]

        def row(rr, _):
            for j in range(8):
                sl = pl.ds(j * 16, 16)
                bh[rr, sl] = jnp.maximum(bh[rr, sl] + bf[rr, sl], 0.0)
            return 0
        lax.fori_loop(0, CE, row, 0)

    iissue(0, 0)
    iissue(1, 1)
    iissue(2, 2)
    iwait(0)
    gissue(0, 0)

    def body(kk, _):
        for u in range(4):
            k = kk * 4 + u
            p = u % 2
            np_ = (u + 1) % 2
            r = u
            nr = (u + 1) % 4
            pr = (u + 3) % 4

            @pl.when(k + 1 < NCH)
            def _():
                iwait(nr)

            @pl.when(k >= 1)
            def _():
                swait(np_)

            @pl.when(k + 1 < NCH)
            def _():
                gissue(np_, nr)

            @pl.when(k + 3 < NCH)
            def _():
                iissue(pr, k + 3)
            gwait(p)
            compute(p)
            sissue(p, r)
        return 0
    lax.fori_loop(0, NCH // 4, body, 0)
    swait((NCH - 1) % 2)

    plsc.subcore_barrier()
    for i in range(NROWS_T // NZB):
        pltpu.sync_copy(acc.at[pl.ds(sid * NROWS_T + i * NZB, NZB)],
                        aggr_hbm.at[cid, pl.ds(sid * NROWS_T + i * NZB, NZB)])


_edge_kernel = pl.kernel(
    _edge_body,
    out_type=[jax.ShapeDtypeStruct((NC, NP, D), jnp.float32)],
    mesh=_mesh,
    scratch_types=([pltpu.VMEM((CE, D), jnp.float32)] * 4
                   + [pltpu.VMEM((CE,), jnp.int32)] * 12
                   + [pltpu.SemaphoreType.DMA] * 8
                   + [pltpu.VMEM_SHARED((NP, D), jnp.float32)]))


def _stats_step(accum, t, i, g_ref, bb_ref, aff_ref):
    rows = lax.broadcasted_iota(jnp.int32, (BR, 1), 0) + i * BR
    mask = rows < N
    tm = jnp.where(mask, t, 0.0)
    nb = jnp.minimum(N - i * BR, BR).astype(jnp.float32)
    bmean = tm.sum(axis=0, keepdims=True) / nb
    dctr = jnp.where(mask, t - bmean, 0.0)
    bm2 = (dctr * dctr).sum(axis=0, keepdims=True)

    @pl.when(i == 0)
    def _():
        accum[...] = jnp.zeros_like(accum)

    n = accum[0:1, :]
    mean = accum[1:2, :]
    n_new = n + nb
    delta = bmean - mean
    accum[0:1, :] = n_new
    accum[1:2, :] = mean + delta * (nb / n_new)
    accum[2:3, :] += bm2 + delta * delta * (n * nb / n_new)

    @pl.when(i == NBLK - 1)
    def _():
        mu = accum[1:2, :]
        var = accum[2:3, :] / N
        rstd = lax.rsqrt(var + 1e-5)
        scale = rstd * g_ref[...]
        shift = bb_ref[...] - mu * scale
        aff_ref[0:1, :] = scale
        aff_ref[1:2, :] = shift


def _m1_body(eps_ref, hl_ref, ag_ref, w_ref, b_ref, g_ref, bb_ref,
             t1_ref, aff_ref, accum):
    i = pl.program_id(0)
    eps = eps_ref[0, 0]
    t0 = (1.0 + eps) * hl_ref[...] + ag_ref[0] + ag_ref[1]
    t1 = jnp.dot(t0, w_ref[...], preferred_element_type=jnp.float32) + b_ref[...]
    t1_ref[...] = t1
    _stats_step(accum, t1, i, g_ref, bb_ref, aff_ref)


def _m1(eps, hl, aggr, w1t, b1, bng, bnb, dout):
    return pl.pallas_call(
        _m1_body,
        grid=(NBLK,),
        in_specs=[
            pl.BlockSpec(memory_space=pltpu.SMEM),
            pl.BlockSpec((BR, D), lambda i: (i, 0)),
            pl.BlockSpec((2, BR, D), lambda i: (0, i, 0)),
            pl.BlockSpec((D, dout), lambda i: (0, 0)),
            pl.BlockSpec((1, dout), lambda i: (0, 0)),
            pl.BlockSpec((1, dout), lambda i: (0, 0)),
            pl.BlockSpec((1, dout), lambda i: (0, 0)),
        ],
        out_specs=[
            pl.BlockSpec((BR, dout), lambda i: (i, 0)),
            pl.BlockSpec((2, dout), lambda i: (0, 0)),
        ],
        out_shape=[
            jax.ShapeDtypeStruct((NP, dout), jnp.float32),
            jax.ShapeDtypeStruct((2, dout), jnp.float32),
        ],
        scratch_shapes=[pltpu.VMEM((3, dout), jnp.float32)],
    )(eps, hl, aggr, w1t, b1, bng, bnb)


def _m2_body(t1_ref, aff_ref, w_ref, b_ref, g_ref, bb_ref,
             t2_ref, aff2_ref, accum):
    i = pl.program_id(0)
    t1n = jnp.maximum(t1_ref[...] * aff_ref[0:1, :] + aff_ref[1:2, :], 0.0)
    t2 = jnp.dot(t1n, w_ref[...], preferred_element_type=jnp.float32) + b_ref[...]
    t2_ref[...] = t2
    _stats_step(accum, t2, i, g_ref, bb_ref, aff2_ref)


def _m2(t1, aff1, w2t, b2, bng, bnb, din, dout):
    return pl.pallas_call(
        _m2_body,
        grid=(NBLK,),
        in_specs=[
            pl.BlockSpec((BR, din), lambda i: (i, 0)),
            pl.BlockSpec((2, din), lambda i: (0, 0)),
            pl.BlockSpec((din, dout), lambda i: (0, 0)),
            pl.BlockSpec((1, dout), lambda i: (0, 0)),
            pl.BlockSpec((1, dout), lambda i: (0, 0)),
            pl.BlockSpec((1, dout), lambda i: (0, 0)),
        ],
        out_specs=[
            pl.BlockSpec((BR, dout), lambda i: (i, 0)),
            pl.BlockSpec((2, dout), lambda i: (0, 0)),
        ],
        out_shape=[
            jax.ShapeDtypeStruct((NP, dout), jnp.float32),
            jax.ShapeDtypeStruct((2, dout), jnp.float32),
        ],
        scratch_shapes=[pltpu.VMEM((3, dout), jnp.float32)],
    )(t1, aff1, w2t, b2, bng, bnb)


def _bn_block(t, g, b):
    mu = t.mean(axis=0, keepdims=True)
    d = t - mu
    var = (d * d).mean(axis=0, keepdims=True)
    return d * lax.rsqrt(var + 1e-5) * g + b


def _vn_body(p_ref, vne_ref, w1_ref, b1_ref, g1_ref, bb1_ref,
             w2_ref, b2_ref, g2_ref, bb2_ref, out_ref):
    v = p_ref[0, :G, :] + p_ref[1, :G, :] + vne_ref[...]
    t = jnp.dot(v, w1_ref[...], preferred_element_type=jnp.float32) + b1_ref[...]
    t = jnp.maximum(_bn_block(t, g1_ref[...], bb1_ref[...]), 0.0)
    t = jnp.dot(t, w2_ref[...], preferred_element_type=jnp.float32) + b2_ref[...]
    t = jnp.maximum(_bn_block(t, g2_ref[...], bb2_ref[...]), 0.0)
    out_ref[...] = t


def _vn(pooled, vne, w1t, b1, g1, bb1, w2t, b2, g2, bb2):
    return pl.pallas_call(
        _vn_body,
        out_shape=jax.ShapeDtypeStruct((G, D), jnp.float32),
    )(pooled, vne, w1t, b1, g1, bb1, w2t, b2, g2, bb2)


def _final_body(hg_ref, cnt_ref, aff_ref, w_ref, b_ref, out_ref):
    hs = hg_ref[0, :G, :] + hg_ref[1, :G, :]
    hs = hs * aff_ref[0:1, :] + aff_ref[1:2, :] * cnt_ref[...]
    cnt = cnt_ref[...]
    hgm = hs / jnp.maximum(cnt, 1.0)
    out_ref[...] = (jnp.dot(hgm, w_ref[...], preferred_element_type=jnp.float32)
                    + b_ref[...])


def _final(hg, cnt, aff, cwt, cb):
    return pl.pallas_call(
        _final_body,
        out_shape=jax.ShapeDtypeStruct((G, T), jnp.float32),
    )(hg, cnt, aff, cwt, cb)


def kernel(x, edge_index, edge_attr, batch, conv_W1, conv_b1, conv_bn_g,
           conv_bn_b, conv_W2, conv_b2, conv_eps, bond_tab, node_bn_g,
           node_bn_b, vn_W1, vn_b1, vn_bn1_g, vn_bn1_b, vn_W2, vn_b2,
           vn_bn2_g, vn_bn2_b, vn_emb, causal_W, causal_b):
    f32 = jnp.float32
    src = edge_index[0]
    dst = edge_index[1]
    srcp = jnp.concatenate([src, jnp.full((EP - E,), N, jnp.int32)])
    dstp = jnp.concatenate([dst, jnp.full((EP - E,), N, jnp.int32)])
    eidx = edge_attr[:, 0] * 25 + edge_attr[:, 1] * 5 + edge_attr[:, 2]
    eixp = jnp.concatenate([eidx, jnp.full((EP - E,), 125, jnp.int32)])
    src3 = srcp.reshape(NW, NCH, CE)
    dst3 = dstp.reshape(NW, NCH, CE)
    eix3 = eixp.reshape(NW, NCH, CE)

    bg = jnp.concatenate([batch, jnp.zeros((NP - N,), jnp.int32)])
    bp = jnp.concatenate([batch, jnp.full((NP - N,), G, jnp.int32)])
    xp = jnp.pad(x, ((0, NP - N), (0, 0)))

    ft = (bond_tab[:, 0][:, :, None, None, :]
          + bond_tab[:, 1][:, None, :, None, :]
          + bond_tab[:, 2][:, None, None, :, :]).reshape(L, 125, D)
    ftab = jnp.pad(ft, ((0, 0), (0, 3), (0, 0)))

    w1t = conv_W1.transpose(0, 2, 1)
    w2t = conv_W2.transpose(0, 2, 1)
    vw1t = vn_W1.transpose(0, 2, 1)
    vw2t = vn_W2.transpose(0, 2, 1)
    cwt = causal_W.T

    a0 = _make_node_kernel(affine=False, relu=False, vne_gather=False,
                           vne_bcast=True, pool=True, write_hl=True,
                           counts=True)
    a_mid = _make_node_kernel(affine=True, relu=True, vne_gather=True,
                              vne_bcast=False, pool=True, write_hl=True,
                              counts=False)
    a_last = _make_node_kernel(affine=True, relu=True, vne_gather=True,
                               vne_bcast=False, pool=False, write_hl=True,
                               counts=False)
    p_k = _make_node_kernel(affine=False, relu=False, vne_gather=False,
                            vne_bcast=False, pool=True, write_hl=False,
                            counts=False)

    hl0, pooled0, counts2 = a0(xp, vn_emb, bp)
    cnt = (counts2[0, :G, 0:1] + counts2[1, :G, 0:1])

    vne = jnp.tile(vn_emb, (G, 1))

    hl = hl0
    pooled = pooled0
    t2 = None
    aff2 = None
    for l in range(L):
        aggr = _edge_kernel(hl, ftab[l], src3, dst3, eix3)[0]
        t1, aff1 = _m1(conv_eps[l].reshape(1, 1), hl, aggr, w1t[l],
                       conv_b1[l].reshape(1, 2 * D),
                       conv_bn_g[l].reshape(1, 2 * D),
                       conv_bn_b[l].reshape(1, 2 * D), 2 * D)
        t2, aff2 = _m2(t1, aff1, w2t[l], conv_b2[l].reshape(1, D),
                       node_bn_g[l].reshape(1, D),
                       node_bn_b[l].reshape(1, D), 2 * D, D)
        if l < L - 1:
            vne = _vn(pooled, vne, vw1t[l], vn_b1[l].reshape(1, 2 * D),
                      vn_bn1_g[l].reshape(1, 2 * D),
                      vn_bn1_b[l].reshape(1, 2 * D),
                      vw2t[l], vn_b2[l].reshape(1, D),
                      vn_bn2_g[l].reshape(1, D), vn_bn2_b[l].reshape(1, D))
            if l < L - 2:
                hl, pooled = a_mid(t2, aff2, vne, bg, bp)
            else:
                hl, = a_last(t2, aff2, vne, bg)

    hg, = p_k(t2, bp)
    return _final(hg, cnt, aff2, cwt, causal_b.reshape(1, T))


# speedup vs baseline: 1.0319x; 1.0319x over previous
import functools

import jax
import jax.numpy as jnp
from jax import lax
from jax.experimental import pallas as pl
from jax.experimental.pallas import tpu as pltpu
from jax.experimental.pallas import tpu_sc as plsc

N = 10000
E = 320000
D = 128
L = 3
G = 512
T = 128

NC = 2
NS = 16
NW = NC * NS

NP = 10240
EP = 327680
GP = 640
NWROWS = NP // NW
EW = EP // NW
CN = 80
CE = 80
GROWS = GP // NS
NROWS_T = NP // NS

BR = 512
NBLK = NP // BR

_mesh = plsc.VectorSubcoreMesh(core_axis_name="c", subcore_axis_name="s")


def _zero_rows(buf, nrows):
    def z(r, _):
        for j in range(8):
            buf[r, pl.ds(j * 16, 16)] = jnp.zeros((16,), jnp.float32)
        return 0
    lax.fori_loop(0, nrows, z, 0)


def _make_node_kernel(affine, relu, vne_gather, vne_bcast, pool, write_hl,
                      counts):
    out_type = []
    if write_hl:
        out_type.append(jax.ShapeDtypeStruct((NP, D), jnp.float32))
    if pool:
        out_type.append(jax.ShapeDtypeStruct((NC, GP, D), jnp.float32))
    if counts:
        out_type.append(jax.ShapeDtypeStruct((NC, GP, D), jnp.float32))

    scratch = [pltpu.VMEM((CN, D), jnp.float32),
               pltpu.VMEM((CN, D), jnp.float32),
               pltpu.VMEM((CN,), jnp.int32),
               pltpu.VMEM((CN,), jnp.int32),
               pltpu.VMEM((2, D), jnp.float32),
               pltpu.VMEM((1, D), jnp.float32),
               pltpu.VMEM((CN, D), jnp.float32),
               pltpu.SemaphoreType.DMA,
               pltpu.SemaphoreType.DMA]
    if pool:
        scratch.append(pltpu.VMEM_SHARED((GP, D), jnp.float32))
    if counts:
        scratch.append(pltpu.VMEM_SHARED((GP, D), jnp.float32))

    def body(*refs):
        it = iter(refs)
        t_hbm = next(it)
        aff_hbm = next(it) if affine else None
        vne_hbm = next(it) if (vne_gather or vne_bcast) else None
        bg_hbm = next(it) if vne_gather else None
        bp_hbm = next(it) if (pool or counts) else None
        hl_hbm = next(it) if write_hl else None
        pooled_hbm = next(it) if pool else None
        counts_hbm = next(it) if counts else None
        buf = next(it); buf2 = next(it); bgv = next(it); bpv = next(it)
        affv = next(it); vnev = next(it); onesv = next(it)
        sem = next(it); sem2 = next(it)
        acc = next(it) if pool else None
        acc2 = next(it) if counts else None

        cid = lax.axis_index("c")
        sid = lax.axis_index("s")
        wid = sid * NC + cid

        if pool or counts:
            _zero_rows(buf, GROWS)
            if pool:
                pltpu.sync_copy(buf.at[pl.ds(0, GROWS)],
                                acc.at[pl.ds(sid * GROWS, GROWS)])
            if counts:
                pltpu.sync_copy(buf.at[pl.ds(0, GROWS)],
                                acc2.at[pl.ds(sid * GROWS, GROWS)])

                def o(r, _):
                    for j in range(8):
                        onesv[r, pl.ds(j * 16, 16)] = jnp.ones((16,),
                                                               jnp.float32)
                    return 0
                lax.fori_loop(0, CN, o, 0)
            plsc.subcore_barrier()

        if affine:
            pltpu.sync_copy(aff_hbm, affv)
        if vne_bcast:
            pltpu.sync_copy(vne_hbm, vnev)

        scale_v = ([affv[0, pl.ds(j * 16, 16)] for j in range(8)]
                   if affine else None)
        shift_v = ([affv[1, pl.ds(j * 16, 16)] for j in range(8)]
                   if affine else None)
        vrow_v = ([vnev[0, pl.ds(j * 16, 16)] for j in range(8)]
                  if vne_bcast else None)

        def chunk(k, _):
            base = wid * NWROWS + k * CN
            pltpu.sync_copy(t_hbm.at[pl.ds(base, CN)], buf)
            if vne_gather:
                pltpu.sync_copy(bg_hbm.at[pl.ds(base, CN)], bgv)
            if pool or counts:
                pltpu.sync_copy(bp_hbm.at[pl.ds(base, CN)], bpv)
            if vne_gather:
                pltpu.async_copy(vne_hbm.at[bgv], buf2, sem).wait()

            def row(r, _):
                for j in range(8):
                    sl = pl.ds(j * 16, 16)
                    v = buf[r, sl]
                    if affine:
                        v = v * scale_v[j] + shift_v[j]
                    if relu:
                        v = jnp.maximum(v, 0.0)
                    if vne_bcast:
                        v = v + vrow_v[j]
                    if vne_gather:
                        v = v + buf2[r, sl]
                    buf[r, sl] = v
                return 0
            lax.fori_loop(0, CN, row, 0)

            if write_hl:
                pltpu.sync_copy(buf, hl_hbm.at[pl.ds(base, CN)])
            if pool:
                pltpu.sync_copy(buf, acc.at[bpv], add=True)
            if counts:
                pltpu.sync_copy(onesv, acc2.at[bpv], add=True)
            return 0
        lax.fori_loop(0, NWROWS // CN, chunk, 0)

        if pool or counts:
            plsc.subcore_barrier()
        if pool:
            pltpu.sync_copy(acc.at[pl.ds(sid * GROWS, GROWS)],
                            pooled_hbm.at[cid, pl.ds(sid * GROWS, GROWS)])
        if counts:
            pltpu.sync_copy(acc2.at[pl.ds(sid * GROWS, GROWS)],
                            counts_hbm.at[cid, pl.ds(sid * GROWS, GROWS)])

    return pl.kernel(body, out_type=out_type, mesh=_mesh,
                     scratch_types=scratch)


NCH = EW // CE
NZB = 128


def _edge_body(hl_hbm, ftab_hbm, src_hbm, dst_hbm, eix_hbm, aggr_hbm,
               bh0, bh1, bf0, bf1,
               sv0, sv1, sv2, sv3, dv0, dv1, dv2, dv3, ev0, ev1, ev2, ev3,
               g0, g1, s0, s1, i0, i1, i2, i3, acc):
    cid = lax.axis_index("c")
    sid = lax.axis_index("s")
    wid = sid * NC + cid

    bufh = (bh0, bh1)
    buff = (bf0, bf1)
    svs = (sv0, sv1, sv2, sv3)
    dvs = (dv0, dv1, dv2, dv3)
    evs = (ev0, ev1, ev2, ev3)
    gsem = (g0, g1)
    ssem = (s0, s1)
    isem = (i0, i1, i2, i3)

    _zero_rows(bh0, CE)
    for i in range(NROWS_T // CE):
        pltpu.sync_copy(bh0, acc.at[pl.ds(sid * NROWS_T + i * CE, CE)])
    plsc.subcore_barrier()

    def iissue(r, k):
        pltpu.async_copy(src_hbm.at[wid, k], svs[r], isem[r])
        pltpu.async_copy(dst_hbm.at[wid, k], dvs[r], isem[r])
        pltpu.async_copy(eix_hbm.at[wid, k], evs[r], isem[r])

    def iwait(r):
        for ref in (svs[r], dvs[r], evs[r]):
            pltpu.make_async_copy(src_hbm.at[wid, 0], ref, isem[r]).wait()

    def gissue(p, r):
        pltpu.async_copy(hl_hbm.at[svs[r]], bufh[p], gsem[p])
        pltpu.async_copy(ftab_hbm.at[evs[r]], buff[p], gsem[p])

    def gwait(p):
        pltpu.make_async_copy(hl_hbm.at[svs[0]], bufh[p], gsem[p]).wait()
        pltpu.make_async_copy(hl_hbm.at[svs[0]], buff[p], gsem[p]).wait()

    def sissue(p, r):
        pltpu.async_copy(bufh[p], acc.at[dvs[r]], ssem[p], add=True)

    def swait(p):
        pltpu.make_async_copy(bufh[p], acc.at[dvs[0]], ssem[p]).wait()

    def compute(p):
        bh, bf = bufh[p], buff[p]

        def row(rr, _):
            for j in range(8):
                sl = pl.ds(j * 16, 16)
                bh[rr, sl] = jnp.maximum(bh[rr, sl] + bf[rr, sl], 0.0)
            return 0
        lax.fori_loop(0, CE, row, 0)

    iissue(0, 0)
    iissue(1, 1)
    iissue(2, 2)
    iwait(0)
    gissue(0, 0)

    def body(kk, _):
        for u in range(4):
            k = kk * 4 + u
            p = u % 2
            np_ = (u + 1) % 2
            r = u
            nr = (u + 1) % 4
            pr = (u + 3) % 4

            @pl.when(k + 1 < NCH)
            def _():
                iwait(nr)

            @pl.when(k >= 1)
            def _():
                swait(np_)

            @pl.when(k + 1 < NCH)
            def _():
                gissue(np_, nr)

            @pl.when(k + 3 < NCH)
            def _():
                iissue(pr, k + 3)
            gwait(p)
            compute(p)
            sissue(p, r)
        return 0
    lax.fori_loop(0, NCH // 4, body, 0)
    swait((NCH - 1) % 2)

    plsc.subcore_barrier()
    for i in range(NROWS_T // NZB):
        pltpu.sync_copy(acc.at[pl.ds(sid * NROWS_T + i * NZB, NZB)],
                        aggr_hbm.at[cid, pl.ds(sid * NROWS_T + i * NZB, NZB)])


_edge_kernel = pl.kernel(
    _edge_body,
    out_type=[jax.ShapeDtypeStruct((NC, NP, D), jnp.float32)],
    mesh=_mesh,
    scratch_types=([pltpu.VMEM((CE, D), jnp.float32)] * 4
                   + [pltpu.VMEM((CE,), jnp.int32)] * 12
                   + [pltpu.SemaphoreType.DMA] * 8
                   + [pltpu.VMEM_SHARED((NP, D), jnp.float32)]))


def _stats_step(accum, t, i, g_ref, bb_ref, aff_ref):
    rows = lax.broadcasted_iota(jnp.int32, (BR, 1), 0) + i * BR
    mask = rows < N
    tm = jnp.where(mask, t, 0.0)
    nb = jnp.minimum(N - i * BR, BR).astype(jnp.float32)
    bmean = tm.sum(axis=0, keepdims=True) / nb
    dctr = jnp.where(mask, t - bmean, 0.0)
    bm2 = (dctr * dctr).sum(axis=0, keepdims=True)

    @pl.when(i == 0)
    def _():
        accum[...] = jnp.zeros_like(accum)

    n = accum[0:1, :]
    mean = accum[1:2, :]
    n_new = n + nb
    delta = bmean - mean
    accum[0:1, :] = n_new
    accum[1:2, :] = mean + delta * (nb / n_new)
    accum[2:3, :] += bm2 + delta * delta * (n * nb / n_new)

    @pl.when(i == NBLK - 1)
    def _():
        mu = accum[1:2, :]
        var = accum[2:3, :] / N
        rstd = lax.rsqrt(var + 1e-5)
        scale = rstd * g_ref[...]
        shift = bb_ref[...] - mu * scale
        aff_ref[0:1, :] = scale
        aff_ref[1:2, :] = shift


def _m1_body(eps_ref, hl_ref, ag_ref, w_ref, b_ref, g_ref, bb_ref,
             t1_ref, aff_ref, accum):
    i = pl.program_id(0)
    eps = eps_ref[0, 0]
    t0 = (1.0 + eps) * hl_ref[...] + ag_ref[0] + ag_ref[1]
    t1 = jnp.dot(t0, w_ref[...], preferred_element_type=jnp.float32) + b_ref[...]
    t1_ref[...] = t1
    _stats_step(accum, t1, i, g_ref, bb_ref, aff_ref)


def _m1(eps, hl, aggr, w1t, b1, bng, bnb, dout):
    return pl.pallas_call(
        _m1_body,
        grid=(NBLK,),
        in_specs=[
            pl.BlockSpec(memory_space=pltpu.SMEM),
            pl.BlockSpec((BR, D), lambda i: (i, 0)),
            pl.BlockSpec((2, BR, D), lambda i: (0, i, 0)),
            pl.BlockSpec((D, dout), lambda i: (0, 0)),
            pl.BlockSpec((1, dout), lambda i: (0, 0)),
            pl.BlockSpec((1, dout), lambda i: (0, 0)),
            pl.BlockSpec((1, dout), lambda i: (0, 0)),
        ],
        out_specs=[
            pl.BlockSpec((BR, dout), lambda i: (i, 0)),
            pl.BlockSpec((2, dout), lambda i: (0, 0)),
        ],
        out_shape=[
            jax.ShapeDtypeStruct((NP, dout), jnp.float32),
            jax.ShapeDtypeStruct((2, dout), jnp.float32),
        ],
        scratch_shapes=[pltpu.VMEM((3, dout), jnp.float32)],
    )(eps, hl, aggr, w1t, b1, bng, bnb)


def _m2_body(t1_ref, aff_ref, w_ref, b_ref, g_ref, bb_ref,
             t2_ref, aff2_ref, accum):
    i = pl.program_id(0)
    t1n = jnp.maximum(t1_ref[...] * aff_ref[0:1, :] + aff_ref[1:2, :], 0.0)
    t2 = jnp.dot(t1n, w_ref[...], preferred_element_type=jnp.float32) + b_ref[...]
    t2_ref[...] = t2
    _stats_step(accum, t2, i, g_ref, bb_ref, aff2_ref)


def _m2(t1, aff1, w2t, b2, bng, bnb, din, dout):
    return pl.pallas_call(
        _m2_body,
        grid=(NBLK,),
        in_specs=[
            pl.BlockSpec((BR, din), lambda i: (i, 0)),
            pl.BlockSpec((2, din), lambda i: (0, 0)),
            pl.BlockSpec((din, dout), lambda i: (0, 0)),
            pl.BlockSpec((1, dout), lambda i: (0, 0)),
            pl.BlockSpec((1, dout), lambda i: (0, 0)),
            pl.BlockSpec((1, dout), lambda i: (0, 0)),
        ],
        out_specs=[
            pl.BlockSpec((BR, dout), lambda i: (i, 0)),
            pl.BlockSpec((2, dout), lambda i: (0, 0)),
        ],
        out_shape=[
            jax.ShapeDtypeStruct((NP, dout), jnp.float32),
            jax.ShapeDtypeStruct((2, dout), jnp.float32),
        ],
        scratch_shapes=[pltpu.VMEM((3, dout), jnp.float32)],
    )(t1, aff1, w2t, b2, bng, bnb)


def _bn_block(t, g, b):
    mu = t.mean(axis=0, keepdims=True)
    d = t - mu
    var = (d * d).mean(axis=0, keepdims=True)
    return d * lax.rsqrt(var + 1e-5) * g + b


def _vn_body(p_ref, vne_ref, w1_ref, b1_ref, g1_ref, bb1_ref,
             w2_ref, b2_ref, g2_ref, bb2_ref, out_ref):
    v = p_ref[0, :G, :] + p_ref[1, :G, :] + vne_ref[...]
    t = jnp.dot(v, w1_ref[...], preferred_element_type=jnp.float32) + b1_ref[...]
    t = jnp.maximum(_bn_block(t, g1_ref[...], bb1_ref[...]), 0.0)
    t = jnp.dot(t, w2_ref[...], preferred_element_type=jnp.float32) + b2_ref[...]
    t = jnp.maximum(_bn_block(t, g2_ref[...], bb2_ref[...]), 0.0)
    out_ref[...] = t


def _vn(pooled, vne, w1t, b1, g1, bb1, w2t, b2, g2, bb2):
    return pl.pallas_call(
        _vn_body,
        out_shape=jax.ShapeDtypeStruct((G, D), jnp.float32),
    )(pooled, vne, w1t, b1, g1, bb1, w2t, b2, g2, bb2)


def _final_body(hg_ref, cnt_ref, aff_ref, w_ref, b_ref, out_ref):
    hs = hg_ref[0, :G, :] + hg_ref[1, :G, :]
    hs = hs * aff_ref[0:1, :] + aff_ref[1:2, :] * cnt_ref[...]
    cnt = cnt_ref[...]
    hgm = hs / jnp.maximum(cnt, 1.0)
    out_ref[...] = (jnp.dot(hgm, w_ref[...], preferred_element_type=jnp.float32)
                    + b_ref[...])


def _final(hg, cnt, aff, cwt, cb):
    return pl.pallas_call(
        _final_body,
        out_shape=jax.ShapeDtypeStruct((G, T), jnp.float32),
    )(hg, cnt, aff, cwt, cb)


def kernel(x, edge_index, edge_attr, batch, conv_W1, conv_b1, conv_bn_g,
           conv_bn_b, conv_W2, conv_b2, conv_eps, bond_tab, node_bn_g,
           node_bn_b, vn_W1, vn_b1, vn_bn1_g, vn_bn1_b, vn_W2, vn_b2,
           vn_bn2_g, vn_bn2_b, vn_emb, causal_W, causal_b):
    f32 = jnp.float32
    src = edge_index[0]
    dst = edge_index[1]
    srcp = jnp.concatenate([src, jnp.full((EP - E,), N, jnp.int32)])
    dstp = jnp.concatenate([dst, jnp.full((EP - E,), N, jnp.int32)])
    eidx = edge_attr[:, 0] * 25 + edge_attr[:, 1] * 5 + edge_attr[:, 2]
    eixp = jnp.concatenate([eidx, jnp.full((EP - E,), 125, jnp.int32)])
    src3 = srcp.reshape(NW, NCH, CE)
    dst3 = dstp.reshape(NW, NCH, CE)
    eix3 = eixp.reshape(NW, NCH, CE)

    bg = jnp.concatenate([batch, jnp.zeros((NP - N,), jnp.int32)])
    bp = jnp.concatenate([batch, jnp.full((NP - N,), G, jnp.int32)])
    xp = jnp.pad(x, ((0, NP - N), (0, 0)))

    ft = (bond_tab[:, 0][:, :, None, None, :]
          + bond_tab[:, 1][:, None, :, None, :]
          + bond_tab[:, 2][:, None, None, :, :]).reshape(L, 125, D)
    ftab = jnp.pad(ft, ((0, 0), (0, 3), (0, 0)))

    w1t = conv_W1.transpose(0, 2, 1)
    w2t = conv_W2.transpose(0, 2, 1)
    vw1t = vn_W1.transpose(0, 2, 1)
    vw2t = vn_W2.transpose(0, 2, 1)
    cwt = causal_W.T

    a0 = _make_node_kernel(affine=False, relu=False, vne_gather=False,
                           vne_bcast=True, pool=True, write_hl=True,
                           counts=True)
    a_mid = _make_node_kernel(affine=True, relu=True, vne_gather=True,
                              vne_bcast=False, pool=True, write_hl=True,
                              counts=False)
    a_last = _make_node_kernel(affine=True, relu=True, vne_gather=True,
                               vne_bcast=False, pool=False, write_hl=True,
                               counts=False)
    p_k = _make_node_kernel(affine=False, relu=False, vne_gather=False,
                            vne_bcast=False, pool=True, write_hl=False,
                            counts=False)

    hl0, pooled0, counts2 = a0(xp, vn_emb, bp)
    cnt = (counts2[0, :G, 0:1] + counts2[1, :G, 0:1])

    vne = jnp.tile(vn_emb, (G, 1))

    hl = hl0
    pooled = pooled0
    t2 = None
    aff2 = None
    for l in range(L):
        aggr = _edge_kernel(hl, ftab[l], src3, dst3, eix3)[0]
        t1, aff1 = _m1(conv_eps[l].reshape(1, 1), hl, aggr, w1t[l],
                       conv_b1[l].reshape(1, 2 * D),
                       conv_bn_g[l].reshape(1, 2 * D),
                       conv_bn_b[l].reshape(1, 2 * D), 2 * D)
        t2, aff2 = _m2(t1, aff1, w2t[l], conv_b2[l].reshape(1, D),
                       node_bn_g[l].reshape(1, D),
                       node_bn_b[l].reshape(1, D), 2 * D, D)
        if l < L - 1:
            vne = _vn(pooled, vne, vw1t[l], vn_b1[l].reshape(1, 2 * D),
                      vn_bn1_g[l].reshape(1, 2 * D),
                      vn_bn1_b[l].reshape(1, 2 * D),
                      vw2t[l], vn_b2[l].reshape(1, D),
                      vn_bn2_g[l].reshape(1, D), vn_bn2_b[l].reshape(1, D))
            if l < L - 2:
                hl, pooled = a_mid(t2, aff2, vne, bg, bp)
            else:
                hl, = a_last(t2, aff2, vne, bg)

    hg, = p_k(t2, bp)
    return _final(hg, cnt, aff2, cwt, causal_b.reshape(1, T))
